# baseline (speedup 1.0000x reference)
import functools

import jax
import jax.numpy as jnp
from jax import lax
from jax.experimental import pallas as pl
from jax.experimental.pallas import tpu as pltpu
from jax.experimental.pallas import tpu_sc as plsc

VOCAB = 1000000
EMBED_DIM = 16
BATCH = 16384
N_FIELDS = 26

_NUM_IDX = BATCH * N_FIELDS
_NW = 32
_PER_W = _NUM_IDX // _NW
_CHUNK = 512
_NCHUNK = _PER_W // _CHUNK
_GROUPS = VOCAB // 8
_OUT_ROWS = _NUM_IDX * EMBED_DIM // 128
_CH_OUT = _CHUNK * EMBED_DIM // 128


def _sig_kernel(tg_hbm, idx_hbm, out_hbm, idx_v, gidx_v, grp_v, out_v, sem):
    wid = lax.axis_index("s") * 2 + lax.axis_index("c")
    base = wid * _PER_W

    def chunk_body(c, carry):
        start = base + c * _CHUNK
        pltpu.sync_copy(idx_hbm.at[pl.ds(start, _CHUNK)], idx_v)

        def gix_body(s, carry2):
            gidx_v[pl.ds(s * 16, 16)] = idx_v[pl.ds(s * 16, 16)] >> 3
            return carry2

        lax.fori_loop(0, _CHUNK // 16, gix_body, 0)
        pltpu.async_copy(tg_hbm.at[gidx_v], grp_v, sem).wait()

        lanes = jnp.arange(16, dtype=jnp.int32)

        def ext_body(j, carry2):
            i = j * 16
            idxs = idx_v[pl.ds(i, 16)]
            sub16 = (idxs & 7) << 4
            rows = i + lanes
            p0 = rows << 4
            for e in range(EMBED_DIM):
                g = plsc.load_gather(grp_v, [rows, sub16 + e])
                s = 1.0 / (1.0 + jnp.exp(-g))
                p = p0 + e
                plsc.store_scatter(out_v, [p >> 7, p & 127], s)
            return carry2

        lax.fori_loop(0, _CHUNK // 16, ext_body, 0)
        orow = wid * (_PER_W * EMBED_DIM // 128) + c * _CH_OUT
        pltpu.sync_copy(out_v, out_hbm.at[pl.ds(orow, _CH_OUT)])
        return carry

    lax.fori_loop(0, _NCHUNK, chunk_body, 0)


@jax.jit
def _run(tg, xf):
    mesh = plsc.VectorSubcoreMesh(core_axis_name="c", subcore_axis_name="s")
    f = functools.partial(
        pl.kernel,
        mesh=mesh,
        out_type=jax.ShapeDtypeStruct((_OUT_ROWS, 128), jnp.float32),
        scratch_types=[
            pltpu.VMEM((_CHUNK,), jnp.int32),
            pltpu.VMEM((_CHUNK,), jnp.int32),
            pltpu.VMEM((_CHUNK, 128), jnp.float32),
            pltpu.VMEM((_CH_OUT, 128), jnp.float32),
            pltpu.SemaphoreType.DMA,
        ],
        compiler_params=pltpu.CompilerParams(needs_layout_passes=False),
    )(_sig_kernel)
    return f(tg, xf)


def kernel(table, x):
    out = _run(table.reshape(_GROUPS, 128), x.reshape(-1))
    return out.reshape(BATCH, N_FIELDS, EMBED_DIM)


# speedup vs baseline: 1.4867x; 1.4867x over previous
import functools

import jax
import jax.numpy as jnp
from jax import lax
from jax.experimental import pallas as pl
from jax.experimental.pallas import tpu as pltpu
from jax.experimental.pallas import tpu_sc as plsc

VOCAB = 1000000
EMBED_DIM = 16
BATCH = 16384
N_FIELDS = 26

_NW = 32
_B_PER_W = BATCH // _NW
_CB = 64
_NCHUNK = _B_PER_W // _CB


def _sig_kernel(table_hbm, x_hbm, out_hbm, idx2_v, idx_v, rows_v, out_v, sem):
    wid = lax.axis_index("s") * 2 + lax.axis_index("c")
    base = wid * _B_PER_W
    for c in range(_NCHUNK):
        b0 = base + c * _CB
        pltpu.sync_copy(x_hbm.at[pl.ds(b0, _CB), :], idx2_v)

        def repack(bb, carry):
            a = idx2_v[bb, pl.ds(0, 16)]
            b = idx2_v[bb, pl.ds(N_FIELDS - 16, 16)]
            idx_v[pl.ds(bb * N_FIELDS, 16)] = a
            idx_v[pl.ds(bb * N_FIELDS + N_FIELDS - 16, 16)] = b
            return carry

        lax.fori_loop(0, _CB, repack, 0)
        pltpu.async_copy(table_hbm.at[idx_v], rows_v, sem).wait()

        def body(bb, carry):
            j0 = bb * N_FIELDS
            for f in range(N_FIELDS):
                r = rows_v[j0 + f]
                out_v[bb, f] = 1.0 / (1.0 + jnp.exp(-r))
            return carry

        lax.fori_loop(0, _CB, body, 0)
        pltpu.sync_copy(out_v, out_hbm.at[pl.ds(b0, _CB), :, :])


@jax.jit
def _run(table, x):
    mesh = plsc.VectorSubcoreMesh(core_axis_name="c", subcore_axis_name="s")
    f = functools.partial(
        pl.kernel,
        mesh=mesh,
        out_type=jax.ShapeDtypeStruct((BATCH, N_FIELDS, EMBED_DIM), jnp.float32),
        scratch_types=[
            pltpu.VMEM((_CB, N_FIELDS), jnp.int32),
            pltpu.VMEM((_CB * N_FIELDS,), jnp.int32),
            pltpu.VMEM((_CB * N_FIELDS, EMBED_DIM), jnp.float32),
            pltpu.VMEM((_CB, N_FIELDS, EMBED_DIM), jnp.float32),
            pltpu.SemaphoreType.DMA,
        ],
        compiler_params=pltpu.CompilerParams(use_tc_tiling_on_sc=False),
    )(_sig_kernel)
    return f(table, x)


def kernel(table, x):
    return _run(table, x)
